# baseline (speedup 1.0000x reference)
import functools

import jax
import jax.numpy as jnp
from jax import lax
from jax.experimental import pallas as pl
from jax.experimental.pallas import tpu as pltpu
from jax.experimental.pallas import tpu_sc as plsc

_L = 16
_NC, _NS = 2, 16
_NW = _NC * _NS
_ROW = 32768
_CHUNK = 8192
_NCH = 7
_NBUF = 3
_SC_ROWS = 56
_UNROLL = 8
_BIG = 2**31 - 1
_RBLK = 8
_RGRID = 32 // _RBLK


def _scan_chunk(buf):
    U = _UNROLL

    def body(i, carry):
        bs, ss = list(carry[0]), list(carry[1])
        for u in range(U):
            v = buf[pl.ds((i * U + u) * _L, _L)]
            m = v > bs[u]
            bs[u] = jnp.where(m, v, bs[u])
            ss[u] = jnp.where(m, i, ss[u])
        return tuple(bs), tuple(ss)

    init = (tuple(jnp.full((_L,), -jnp.inf, jnp.float32) for _ in range(U)),
            tuple(jnp.zeros((_L,), jnp.int32) for _ in range(U)))
    bs, ss = lax.fori_loop(0, _CHUNK // _L // U, body, init)

    lanes = lax.broadcasted_iota(jnp.int32, (_L,), 0)
    pairs = [(bs[u], (ss[u] << 7) + (u << 4) + lanes) for u in range(U)]
    while len(pairs) > 1:
        nxt = []
        for a in range(0, len(pairs), 2):
            (va, ia), (vb, ib) = pairs[a], pairs[a + 1]
            m = (va > vb) | ((va == vb) & (ia < ib))
            nxt.append((jnp.where(m, va, vb), jnp.where(m, ia, ib)))
        pairs = nxt
    return pairs[0]


def _sc_body(x1_hbm, pval_hbm, pidx_hbm,
             buf0, buf1, buf2, pvb, pib, sem0, sem1, sem2):
    wid = lax.axis_index("s") * _NC + lax.axis_index("c")
    bufs, sems = (buf0, buf1, buf2), (sem0, sem1, sem2)
    handles = [None] * _NCH
    per_row = _ROW // _CHUNK

    def issue(k):
        h = _NCH * wid + k
        row = h // per_row
        col = (h % per_row) * _CHUNK
        handles[k] = pltpu.async_copy(x1_hbm.at[row, pl.ds(col, _CHUNK)],
                                      bufs[k % _NBUF], sems[k % _NBUF])

    for k in range(_NBUF):
        issue(k)
    best = None
    for k in range(_NCH):
        handles[k].wait()
        bc, ci = _scan_chunk(bufs[k % _NBUF])
        if k + _NBUF < _NCH:
            issue(k + _NBUF)
        absi = k * _CHUNK + ci
        if best is None:
            best = (bc, absi)
        else:
            b0, i0 = best
            m = bc > b0
            best = (jnp.where(m, bc, b0), jnp.where(m, absi, i0))

    pvb[...] = best[0]
    pib[...] = wid * (_NCH * _CHUNK) + best[1]
    cv = pltpu.async_copy(pvb, pval_hbm.at[wid], sem0)
    ci_ = pltpu.async_copy(pib, pidx_hbm.at[wid], sem1)
    cv.wait()
    ci_.wait()


_sc_x1_partial = functools.partial(
    pl.kernel,
    out_type=[
        jax.ShapeDtypeStruct((_NW, _L), jnp.float32),
        jax.ShapeDtypeStruct((_NW, _L), jnp.int32),
    ],
    mesh=plsc.VectorSubcoreMesh(core_axis_name="c", subcore_axis_name="s"),
    scratch_types=[
        pltpu.VMEM((_CHUNK,), jnp.float32),
        pltpu.VMEM((_CHUNK,), jnp.float32),
        pltpu.VMEM((_CHUNK,), jnp.float32),
        pltpu.VMEM((_L,), jnp.float32),
        pltpu.VMEM((_L,), jnp.int32),
        pltpu.SemaphoreType.DMA,
        pltpu.SemaphoreType.DMA,
        pltpu.SemaphoreType.DMA,
    ],
)(_sc_body)


def _row_argmax(x):
    m = jnp.max(x, axis=1, keepdims=True)
    idx = lax.broadcasted_iota(jnp.int32, x.shape, 1)
    return jnp.min(jnp.where(x == m, idx, _BIG), axis=1)


def _rows_body(a_ref, b_ref, c_ref, ra_ref, xv_ref, xi_ref):
    i = pl.program_id(0)
    ra = _row_argmax(a_ref[...])
    rb = _row_argmax(b_ref[...])
    for j in range(_RGRID):
        @pl.when(i == j)
        def _():
            ra_ref[j * _RBLK:(j + 1) * _RBLK] = ra
            ra_ref[32 + j * _RBLK:32 + (j + 1) * _RBLK] = rb

    @pl.when(i == 0)
    def _():
        x = c_ref[...]
        xarg = _row_argmax(x)
        xmax = jnp.max(x, axis=1)
        xv_ref[...] = xmax
        xi_ref[...] = (
            (_SC_ROWS + lax.broadcasted_iota(jnp.int32, (8,), 0)) * _ROW
            + xarg)


def _merge_body(pv_ref, pi_ref, xv_ref, xi_ref, ra_ref, o_ref):
    mv = jnp.maximum(jnp.max(pv_ref[...]), jnp.max(xv_ref[...]))
    c1 = jnp.min(jnp.where(pv_ref[...] == mv, pi_ref[...], _BIG))
    c2 = jnp.min(jnp.where(xv_ref[...] == mv, xi_ref[...], _BIG))
    o_ref[...] = ra_ref[...] + jnp.minimum(c1, c2)


def kernel(x1, x2):
    pvals, pidx = _sc_x1_partial(x1)
    rowarg, xv, xi = pl.pallas_call(
        _rows_body,
        grid=(_RGRID,),
        in_specs=[
            pl.BlockSpec((_RBLK, _ROW), lambda i: (i, 0)),
            pl.BlockSpec((_RBLK, _ROW), lambda i: (i + _RGRID, 0)),
            pl.BlockSpec((8, _ROW), lambda i: (_SC_ROWS // 8, 0)),
        ],
        out_specs=[pl.BlockSpec((64,), lambda i: (0,)),
                   pl.BlockSpec((8,), lambda i: (0,)),
                   pl.BlockSpec((8,), lambda i: (0,))],
        out_shape=[jax.ShapeDtypeStruct((64,), jnp.int32),
                   jax.ShapeDtypeStruct((8,), jnp.float32),
                   jax.ShapeDtypeStruct((8,), jnp.int32)],
    )(x2, x2, x1)
    return pl.pallas_call(
        _merge_body,
        out_shape=jax.ShapeDtypeStruct((64,), jnp.int32),
    )(pvals, pidx, xv, xi, rowarg)


# speedup vs baseline: 1.0389x; 1.0389x over previous
import functools

import jax
import jax.numpy as jnp
from jax import lax
from jax.experimental import pallas as pl
from jax.experimental.pallas import tpu as pltpu
from jax.experimental.pallas import tpu_sc as plsc

_L = 16
_NC, _NS = 2, 16
_NW = _NC * _NS
_ROW = 32768
_CHUNK = 16384
_NCH = 3
_NBUF = 3
_SC_ROWS = 48
_UNROLL = 8
_BIG = 2**31 - 1
_RBLK = 8
_RGRID = 32 // _RBLK


def _scan_chunk(buf):
    U = _UNROLL

    def body(i, carry):
        bs, ss = list(carry[0]), list(carry[1])
        for u in range(U):
            v = buf[pl.ds((i * U + u) * _L, _L)]
            m = v > bs[u]
            bs[u] = jnp.where(m, v, bs[u])
            ss[u] = jnp.where(m, i, ss[u])
        return tuple(bs), tuple(ss)

    init = (tuple(jnp.full((_L,), -jnp.inf, jnp.float32) for _ in range(U)),
            tuple(jnp.zeros((_L,), jnp.int32) for _ in range(U)))
    bs, ss = lax.fori_loop(0, _CHUNK // _L // U, body, init)

    lanes = lax.broadcasted_iota(jnp.int32, (_L,), 0)
    pairs = [(bs[u], (ss[u] << 7) + (u << 4) + lanes) for u in range(U)]
    while len(pairs) > 1:
        nxt = []
        for a in range(0, len(pairs), 2):
            (va, ia), (vb, ib) = pairs[a], pairs[a + 1]
            m = (va > vb) | ((va == vb) & (ia < ib))
            nxt.append((jnp.where(m, va, vb), jnp.where(m, ia, ib)))
        pairs = nxt
    return pairs[0]


def _sc_body(x1_hbm, pval_hbm, pidx_hbm,
             buf0, buf1, buf2, pvb, pib, sem0, sem1, sem2):
    wid = lax.axis_index("s") * _NC + lax.axis_index("c")
    bufs, sems = (buf0, buf1, buf2), (sem0, sem1, sem2)
    handles = [None] * _NCH
    per_row = _ROW // _CHUNK

    def issue(k):
        h = _NCH * wid + k
        row = h // per_row
        col = (h % per_row) * _CHUNK
        handles[k] = pltpu.async_copy(x1_hbm.at[row, pl.ds(col, _CHUNK)],
                                      bufs[k % _NBUF], sems[k % _NBUF])

    for k in range(_NBUF):
        issue(k)
    best = None
    for k in range(_NCH):
        handles[k].wait()
        bc, ci = _scan_chunk(bufs[k % _NBUF])
        if k + _NBUF < _NCH:
            issue(k + _NBUF)
        absi = k * _CHUNK + ci
        if best is None:
            best = (bc, absi)
        else:
            b0, i0 = best
            m = bc > b0
            best = (jnp.where(m, bc, b0), jnp.where(m, absi, i0))

    pvb[...] = best[0]
    pib[...] = wid * (_NCH * _CHUNK) + best[1]
    cv = pltpu.async_copy(pvb, pval_hbm.at[wid], sem0)
    ci_ = pltpu.async_copy(pib, pidx_hbm.at[wid], sem1)
    cv.wait()
    ci_.wait()


_sc_x1_partial = functools.partial(
    pl.kernel,
    out_type=[
        jax.ShapeDtypeStruct((_NW, _L), jnp.float32),
        jax.ShapeDtypeStruct((_NW, _L), jnp.int32),
    ],
    mesh=plsc.VectorSubcoreMesh(core_axis_name="c", subcore_axis_name="s"),
    scratch_types=[
        pltpu.VMEM((_CHUNK,), jnp.float32),
        pltpu.VMEM((_CHUNK,), jnp.float32),
        pltpu.VMEM((_CHUNK,), jnp.float32),
        pltpu.VMEM((_L,), jnp.float32),
        pltpu.VMEM((_L,), jnp.int32),
        pltpu.SemaphoreType.DMA,
        pltpu.SemaphoreType.DMA,
        pltpu.SemaphoreType.DMA,
    ],
)(_sc_body)


def _row_argmax(x):
    m = jnp.max(x, axis=1, keepdims=True)
    idx = lax.broadcasted_iota(jnp.int32, x.shape, 1)
    return jnp.min(jnp.where(x == m, idx, _BIG), axis=1)


def _rows_body(a_ref, b_ref, c_ref, ra_ref, xv_ref, xi_ref):
    i = pl.program_id(0)
    ra = _row_argmax(a_ref[...])
    rb = _row_argmax(b_ref[...])
    for j in range(_RGRID):
        @pl.when(i == j)
        def _():
            ra_ref[j * _RBLK:(j + 1) * _RBLK] = ra
            ra_ref[32 + j * _RBLK:32 + (j + 1) * _RBLK] = rb

    @pl.when(i < 2)
    def _():
        x = c_ref[...]
        xarg = _row_argmax(x)
        xmax = jnp.max(x, axis=1)
        for j in range(2):
            @pl.when(i == j)
            def _():
                base = _SC_ROWS + j * 8
                xv_ref[j * 8:(j + 1) * 8] = xmax
                xi_ref[j * 8:(j + 1) * 8] = (
                    (base + lax.broadcasted_iota(jnp.int32, (8,), 0)) * _ROW
                    + xarg)


def _merge_body(pv_ref, pi_ref, xv_ref, xi_ref, ra_ref, o_ref):
    mv = jnp.maximum(jnp.max(pv_ref[...]), jnp.max(xv_ref[...]))
    c1 = jnp.min(jnp.where(pv_ref[...] == mv, pi_ref[...], _BIG))
    c2 = jnp.min(jnp.where(xv_ref[...] == mv, xi_ref[...], _BIG))
    o_ref[...] = ra_ref[...] + jnp.minimum(c1, c2)


def kernel(x1, x2):
    pvals, pidx = _sc_x1_partial(x1)
    rowarg, xv, xi = pl.pallas_call(
        _rows_body,
        grid=(_RGRID,),
        in_specs=[
            pl.BlockSpec((_RBLK, _ROW), lambda i: (i, 0)),
            pl.BlockSpec((_RBLK, _ROW), lambda i: (i + _RGRID, 0)),
            pl.BlockSpec((8, _ROW),
                         lambda i: (jnp.where(i < 2, _SC_ROWS // 8 + i,
                                              _SC_ROWS // 8 + 1), 0)),
        ],
        out_specs=[pl.BlockSpec((64,), lambda i: (0,)),
                   pl.BlockSpec((16,), lambda i: (0,)),
                   pl.BlockSpec((16,), lambda i: (0,))],
        out_shape=[jax.ShapeDtypeStruct((64,), jnp.int32),
                   jax.ShapeDtypeStruct((16,), jnp.float32),
                   jax.ShapeDtypeStruct((16,), jnp.int32)],
    )(x2, x2, x1)
    return pl.pallas_call(
        _merge_body,
        out_shape=jax.ShapeDtypeStruct((64,), jnp.int32),
    )(pvals, pidx, xv, xi, rowarg)
